# baseline (speedup 1.0000x reference)
import functools

import jax
import jax.numpy as jnp
from jax import lax
from jax.experimental import pallas as pl
from jax.experimental.pallas import tpu as pltpu
from jax.experimental.pallas import tpu_sc as plsc

_B = 4
_S = 8192
_H = 64
_LANES = 16

_info = plsc.get_sparse_core_info()
_NC = _info.num_cores
_NS = _info.num_subcores
_NW = _NC * _NS

_TOK = _B * _S
_TPW = _TOK // _NW
_C = 512
_NCH = _TPW // _C

_mesh = plsc.VectorSubcoreMesh(core_axis_name="c", subcore_axis_name="s")


@functools.partial(
    pl.kernel,
    mesh=_mesh,
    out_type=jax.ShapeDtypeStruct((_TOK, _H), jnp.float32),
    scratch_types=[
        pltpu.VMEM((_TPW,), jnp.int32),
        pltpu.VMEM((_C, _H), jnp.float32),
        pltpu.VMEM((_C, _H), jnp.float32),
        pltpu.SemaphoreType.DMA,
    ],
    compiler_params=pltpu.CompilerParams(use_tc_tiling_on_sc=False),
)
def _embed(ids_hbm, table_hbm, pos_hbm, out_hbm, idx_v, rows_v, pos_v, sem):
    wid = lax.axis_index("s") * _NC + lax.axis_index("c")
    base = wid * _TPW
    pltpu.sync_copy(ids_hbm.at[pl.ds(base, _TPW)], idx_v)
    for ci in range(_NCH):
        off = ci * _C
        gath = pltpu.async_copy(
            table_hbm.at[idx_v.at[pl.ds(off, _C)]], rows_v, sem
        )
        pos_off = lax.rem(base + off, _S)
        pltpu.sync_copy(pos_hbm.at[pl.ds(pos_off, _C)], pos_v)
        gath.wait()

        def add_body(t, carry):
            for j in range(_H // _LANES):
                sl = pl.ds(j * _LANES, _LANES)
                rows_v[t, sl] = rows_v[t, sl] + pos_v[t, sl]
            return carry

        lax.fori_loop(0, _C, add_body, 0)
        pltpu.sync_copy(rows_v, out_hbm.at[pl.ds(base + off, _C)])


def kernel(token_ids, embedding_table, pos_table):
    ids = token_ids.reshape(-1).astype(jnp.int32)
    out = _embed(ids, embedding_table, pos_table)
    return out.reshape(_B, _S, _H)


# speedup vs baseline: 2.0091x; 2.0091x over previous
import functools

import jax
import jax.numpy as jnp
from jax import lax
from jax.experimental import pallas as pl
from jax.experimental.pallas import tpu as pltpu
from jax.experimental.pallas import tpu_sc as plsc

_B = 4
_S = 8192
_H = 64
_V = 1000000
_LANES = 16

_info = plsc.get_sparse_core_info()
_NC = _info.num_cores
_NS = _info.num_subcores
_NW = _NC * _NS

_TOK = _B * _S
_TPW = _TOK // _NW
_NTC = (_V + 127) // 128
_VTAIL = (_NTC - 1) * 128
_NSUP = 8
_GR = 16
_TRASH = _TOK

_mesh = plsc.VectorSubcoreMesh(core_axis_name="c", subcore_axis_name="s")
_params = pltpu.CompilerParams(needs_layout_passes=False)


@functools.partial(
    pl.kernel,
    mesh=_mesh,
    out_type=jax.ShapeDtypeStruct((_TOK * _H,), jnp.float32),
    scratch_types=[
        pltpu.VMEM((_TOK + 16,), jnp.int32),
        pltpu.VMEM((_TOK + 16,), jnp.int32),
        pltpu.VMEM((_TOK + 16,), jnp.int32),
        pltpu.VMEM((2, _H, 128), jnp.float32),
        pltpu.VMEM((_H, 64), jnp.float32),
        pltpu.VMEM((_GR, _H), jnp.float32),
        pltpu.SemaphoreType.DMA,
        pltpu.SemaphoreType.DMA,
    ],
    compiler_params=_params,
)
def _gather_stage(ids_hbm, tab_t_hbm, tail_t_hbm, stage_hbm,
                  ids_v, sup_v, col_v, blk_v, tail_v, ring_v,
                  sem_blk, sem_out):
    wid = lax.axis_index("s") * _NC + lax.axis_index("c")
    iota = lax.iota(jnp.int32, _LANES)
    cw0 = (wid * _NTC) // _NW
    cw1 = ((wid + 1) * _NTC) // _NW
    cw1p = lax.min(cw1, _NTC - 1)
    ncols = cw1p - cw0

    pltpu.sync_copy(ids_hbm, ids_v.at[pl.ds(0, _TOK)])

    def scan_body(k, cnt):
        x = ids_v[pl.ds(k * _LANES, _LANES)]
        c = x >> 7
        m = (c >= cw0) & (c < cw1)
        vloc = x - cw0 * 128
        p = ((iota + _LANES * k) << 15) | vloc
        pfx = plsc.cumsum(m.astype(jnp.int32))
        iidx = jnp.where(m, cnt + pfx - 1, jnp.int32(_TRASH))
        plsc.store_scatter(ids_v, [iidx], p)
        return cnt + lax.reduce_max(pfx, (0,))

    cnt = lax.fori_loop(0, _TOK // _LANES, scan_body, 0)

    def filter_list(src_v, n_src, dst_v, pred):
        def body(q, acc):
            p16 = src_v[pl.ds(q * _LANES, _LANES)]
            valid = (iota + _LANES * q) < n_src
            m = valid & pred(p16 & 0x7FFF)
            pfx = plsc.cumsum(m.astype(jnp.int32))
            iidx = jnp.where(m, acc + pfx - 1, jnp.int32(_TRASH))
            plsc.store_scatter(dst_v, [iidx], p16)
            return acc + lax.reduce_max(pfx, (0,))

        return lax.fori_loop(0, (n_src + _LANES - 1) // _LANES, body, 0)

    def extract_scalar(src_v, i):
        chunk = src_v[pl.ds((i // _LANES) * _LANES, _LANES)]
        lane = lax.rem(i, _LANES)
        return lax.reduce_max(
            jnp.where(iota == lane, chunk, jnp.int32(0)), (0,)
        )

    def fire_stage(src_ref, l, n, g):
        r = lax.rem(g, _GR)

        @pl.when(g >= _GR)
        def _():
            pltpu.make_async_copy(
                stage_hbm.at[pl.ds(0, _H)], ring_v.at[0], sem_out
            ).wait()

        lvec = jnp.full((_LANES,), l, jnp.int32)
        for j in range(_H // _LANES):
            ring_v[r, pl.ds(j * _LANES, _LANES)] = plsc.load_gather(
                src_ref, [iota + j * _LANES, lvec]
            )
        pltpu.async_copy(
            ring_v.at[r], stage_hbm.at[pl.ds(n * _H, _H)], sem_out
        )
        return g + 1

    def fire_blk(h):
        off = pl.multiple_of((cw0 + h) * 128, 128)
        pltpu.async_copy(
            tab_t_hbm.at[:, pl.ds(off, 128)], blk_v.at[lax.rem(h, 2)],
            sem_blk,
        )

    @pl.when(ncols > 0)
    def _():
        fire_blk(0)

    g = 0
    for sc in range(_NSUP):
        n_sc = filter_list(ids_v, cnt, sup_v,
                           lambda vl, sc=sc: (vl >> 12) == sc)

        def col_body(h, g):
            @pl.when(h + 1 < ncols)
            def _():
                fire_blk(h + 1)

            pltpu.make_async_copy(
                tab_t_hbm.at[:, pl.ds(0, 128)], blk_v.at[0], sem_blk
            ).wait()

            n_c = filter_list(sup_v, n_sc, col_v,
                              lambda vl, h=h: (vl >> 7) == h)
            blk = blk_v.at[lax.rem(h, 2)]

            def match_body(i, g):
                p = extract_scalar(col_v, i)
                return fire_stage(blk, p & 0x7F, p >> 15, g)

            return lax.fori_loop(0, n_c, match_body, g)

        lo = lax.min(sc * 32, ncols)
        hi = lax.min((sc + 1) * 32, ncols)
        g = lax.fori_loop(lo, hi, col_body, g)

    @pl.when(cw1 == _NTC)
    def _():
        pltpu.sync_copy(tail_t_hbm, tail_v)

    thresh = (_NTC - 1 - cw0) * 128

    n_t = filter_list(ids_v, jnp.where(cw1 == _NTC, cnt, 0), col_v,
                      lambda vl: vl >= thresh)

    def tail_body(i, g):
        p = extract_scalar(col_v, i)
        return fire_stage(tail_v, (p & 0x7FFF) - thresh, p >> 15, g)

    g = lax.fori_loop(0, n_t, tail_body, g)

    def drain(i, carry):
        pltpu.make_async_copy(
            stage_hbm.at[pl.ds(0, _H)], ring_v.at[0], sem_out
        ).wait()
        return carry

    lax.fori_loop(0, lax.min(g, _GR), drain, 0)


_C = 512


@functools.partial(
    pl.kernel,
    mesh=_mesh,
    out_type=jax.ShapeDtypeStruct((_B, _H, _S), jnp.float32),
    scratch_types=[
        pltpu.VMEM((_C * _H,), jnp.float32),
        pltpu.VMEM((_H, _C), jnp.float32),
        pltpu.VMEM((_H, _C), jnp.float32),
    ],
    compiler_params=_params,
)
def _pos_add(stage_hbm, pos_t_hbm, out_hbm, stg_v, pos_v, rows_v):
    wid = lax.axis_index("s") * _NC + lax.axis_index("c")
    iota = lax.iota(jnp.int32, _LANES)
    base = wid * _TPW
    b = base // _S
    s0 = base - b * _S
    for ci in range(_TPW // _C):
        coff = ci * _C
        pltpu.sync_copy(
            stage_hbm.at[pl.ds((base + coff) * _H, _C * _H)], stg_v
        )
        pltpu.sync_copy(pos_t_hbm.at[:, pl.ds(s0 + coff, _C)], pos_v)

        def add_body(f, carry):
            for t in range(_C // _LANES):
                sl = pl.ds(t * _LANES, _LANES)
                idx = (iota + t * _LANES) * _H + f
                rows_v[f, sl] = plsc.load_gather(stg_v, [idx]) + pos_v[f, sl]
            return carry

        lax.fori_loop(0, _H, add_body, 0)
        pltpu.sync_copy(rows_v, out_hbm.at[b, :, pl.ds(s0 + coff, _C)])


def kernel(token_ids, embedding_table, pos_table):
    ids = token_ids.reshape(-1)
    tail_t = embedding_table[_VTAIL:].T
    stage = _gather_stage(ids, embedding_table.T, tail_t)
    out3 = _pos_add(stage, pos_table.T)
    return out3.transpose(0, 2, 1)


# speedup vs baseline: 2.2506x; 1.1202x over previous
import functools

import jax
import jax.numpy as jnp
from jax import lax
from jax.experimental import pallas as pl
from jax.experimental.pallas import tpu as pltpu
from jax.experimental.pallas import tpu_sc as plsc

_B = 4
_S = 8192
_H = 64
_V = 1000000
_LANES = 16

_info = plsc.get_sparse_core_info()
_NC = _info.num_cores
_NS = _info.num_subcores
_NW = _NC * _NS

_TOK = _B * _S
_TPW = _TOK // _NW
_NTC = (_V + 127) // 128
_VTAIL = (_NTC - 1) * 128
_NSUP = 8
_GR = 16
_TRASH = _TOK
_SSTR = 72

_mesh = plsc.VectorSubcoreMesh(core_axis_name="c", subcore_axis_name="s")
_params = pltpu.CompilerParams(needs_layout_passes=False)


@functools.partial(
    pl.kernel,
    mesh=_mesh,
    out_type=jax.ShapeDtypeStruct((_TOK * _SSTR,), jnp.float32),
    scratch_types=[
        pltpu.VMEM((_TOK + 16,), jnp.int32),
        pltpu.VMEM((_TOK + 16,), jnp.int32),
        pltpu.VMEM((_TOK + 16,), jnp.int32),
        pltpu.VMEM((2, _H, 128), jnp.float32),
        pltpu.VMEM((_H, 64), jnp.float32),
        pltpu.VMEM((_GR, _H), jnp.float32),
        pltpu.SemaphoreType.DMA,
        pltpu.SemaphoreType.DMA,
    ],
    compiler_params=_params,
)
def _gather_stage(ids_hbm, tab_t_hbm, tail_t_hbm, stage_hbm,
                  ids_v, sup_v, col_v, blk_v, tail_v, ring_v,
                  sem_blk, sem_out):
    wid = lax.axis_index("s") * _NC + lax.axis_index("c")
    iota = lax.iota(jnp.int32, _LANES)
    cw0 = (wid * _NTC) // _NW
    cw1 = ((wid + 1) * _NTC) // _NW
    cw1p = lax.min(cw1, _NTC - 1)
    ncols = cw1p - cw0

    pltpu.sync_copy(ids_hbm, ids_v.at[pl.ds(0, _TOK)])

    def scan_body(k, cnt):
        x = ids_v[pl.ds(k * _LANES, _LANES)]
        c = x >> 7
        m = (c >= cw0) & (c < cw1)
        vloc = x - cw0 * 128
        p = ((iota + _LANES * k) << 15) | vloc
        pfx = plsc.cumsum(m.astype(jnp.int32))
        iidx = jnp.where(m, cnt + pfx - 1, jnp.int32(_TRASH))
        plsc.store_scatter(ids_v, [iidx], p)
        return cnt + plsc.all_reduce_population_count(m)[0]

    cnt = lax.fori_loop(0, _TOK // _LANES, scan_body, 0)

    def filter_list(src_v, n_src, dst_v, pred):
        def body(q, acc):
            p16 = src_v[pl.ds(q * _LANES, _LANES)]
            valid = (iota + _LANES * q) < n_src
            m = valid & pred(p16 & 0x7FFF)
            pfx = plsc.cumsum(m.astype(jnp.int32))
            iidx = jnp.where(m, acc + pfx - 1, jnp.int32(_TRASH))
            plsc.store_scatter(dst_v, [iidx], p16)
            return acc + plsc.all_reduce_population_count(m)[0]

        return lax.fori_loop(0, (n_src + _LANES - 1) // _LANES, body, 0)

    def extract_scalar(src_v, i):
        chunk = src_v[pl.ds((i // _LANES) * _LANES, _LANES)]
        lane = lax.rem(i, _LANES)
        return chunk[jnp.full((_LANES,), lane, jnp.int32)][0]

    def fire_stage(src_ref, l, n, g):
        r = lax.rem(g, _GR)

        @pl.when(g >= _GR)
        def _():
            pltpu.make_async_copy(
                stage_hbm.at[pl.ds(0, _H)], ring_v.at[0], sem_out
            ).wait()

        lvec = jnp.full((_LANES,), l, jnp.int32)
        for j in range(_H // _LANES):
            ring_v[r, pl.ds(j * _LANES, _LANES)] = plsc.load_gather(
                src_ref, [iota + j * _LANES, lvec]
            )
        pltpu.async_copy(
            ring_v.at[r], stage_hbm.at[pl.ds(n * _SSTR, _H)], sem_out
        )
        return g + 1

    def fire_blk(h):
        off = pl.multiple_of((cw0 + h) * 128, 128)
        pltpu.async_copy(
            tab_t_hbm.at[:, pl.ds(off, 128)], blk_v.at[lax.rem(h, 2)],
            sem_blk,
        )

    @pl.when(ncols > 0)
    def _():
        fire_blk(0)

    g = 0
    for sc in range(_NSUP):
        n_sc = filter_list(ids_v, cnt, sup_v,
                           lambda vl, sc=sc: (vl >> 12) == sc)

        def col_body(h, g):
            @pl.when(h + 1 < ncols)
            def _():
                fire_blk(h + 1)

            pltpu.make_async_copy(
                tab_t_hbm.at[:, pl.ds(0, 128)], blk_v.at[0], sem_blk
            ).wait()

            n_c = filter_list(sup_v, n_sc, col_v,
                              lambda vl, h=h: (vl >> 7) == h)
            blk = blk_v.at[lax.rem(h, 2)]

            def match_body(i, g):
                p = extract_scalar(col_v, i)
                return fire_stage(blk, p & 0x7F, p >> 15, g)

            return lax.fori_loop(0, n_c, match_body, g)

        lo = lax.min(sc * 32, ncols)
        hi = lax.min((sc + 1) * 32, ncols)
        g = lax.fori_loop(lo, hi, col_body, g)

    @pl.when(cw1 == _NTC)
    def _():
        pltpu.sync_copy(tail_t_hbm, tail_v)

    thresh = (_NTC - 1 - cw0) * 128

    n_t = filter_list(ids_v, jnp.where(cw1 == _NTC, cnt, 0), col_v,
                      lambda vl: vl >= thresh)

    def tail_body(i, g):
        p = extract_scalar(col_v, i)
        return fire_stage(tail_v, (p & 0x7FFF) - thresh, p >> 15, g)

    g = lax.fori_loop(0, n_t, tail_body, g)

    def drain(i, carry):
        pltpu.make_async_copy(
            stage_hbm.at[pl.ds(0, _H)], ring_v.at[0], sem_out
        ).wait()
        return carry

    lax.fori_loop(0, lax.min(g, _GR), drain, 0)


_C = 512


@functools.partial(
    pl.kernel,
    mesh=_mesh,
    out_type=jax.ShapeDtypeStruct((_B, _H, _S), jnp.float32),
    scratch_types=[
        pltpu.VMEM((_C * _SSTR,), jnp.float32),
        pltpu.VMEM((_H, _C), jnp.float32),
        pltpu.VMEM((_H, _C), jnp.float32),
    ],
    compiler_params=_params,
)
def _pos_add(stage_hbm, pos_t_hbm, out_hbm, stg_v, pos_v, rows_v):
    wid = lax.axis_index("s") * _NC + lax.axis_index("c")
    iota = lax.iota(jnp.int32, _LANES)
    base = wid * _TPW
    b = base // _S
    s0 = base - b * _S
    for ci in range(_TPW // _C):
        coff = ci * _C
        pltpu.sync_copy(
            stage_hbm.at[pl.ds((base + coff) * _SSTR, _C * _SSTR)], stg_v
        )
        pltpu.sync_copy(pos_t_hbm.at[:, pl.ds(s0 + coff, _C)], pos_v)

        def add_body(f, carry):
            for t in range(_C // _LANES):
                sl = pl.ds(t * _LANES, _LANES)
                idx = (iota + t * _LANES) * _SSTR + f
                rows_v[f, sl] = plsc.load_gather(stg_v, [idx]) + pos_v[f, sl]
            return carry

        lax.fori_loop(0, _H, add_body, 0)
        pltpu.sync_copy(rows_v, out_hbm.at[b, :, pl.ds(s0 + coff, _C)])


def kernel(token_ids, embedding_table, pos_table):
    ids = token_ids.reshape(-1)
    tail_t = embedding_table[_VTAIL:].T
    stage = _gather_stage(ids, embedding_table.T, tail_t)
    out3 = _pos_add(stage, pos_table.T)
    return out3.transpose(0, 2, 1)


# speedup vs baseline: 2.9823x; 1.3251x over previous
import functools

import jax
import jax.numpy as jnp
from jax import lax
from jax.experimental import pallas as pl
from jax.experimental.pallas import tpu as pltpu
from jax.experimental.pallas import tpu_sc as plsc

_B = 4
_S = 8192
_H = 64
_V = 1000000
_LANES = 16

_info = plsc.get_sparse_core_info()
_NC = _info.num_cores
_NS = _info.num_subcores
_NW = _NC * _NS

_TOK = _B * _S
_TPW = _TOK // _NW
_NTC = (_V + 127) // 128
_VTAIL = (_NTC - 1) * 128
_NSUP = 8
_GR = 8
_TRASH = _TOK
_SSTR = 72
_M = 16384
_BW = 512
_ICH = 2048

_mesh = plsc.VectorSubcoreMesh(core_axis_name="c", subcore_axis_name="s")
_params = pltpu.CompilerParams(needs_layout_passes=False)


@functools.partial(
    pl.kernel,
    mesh=_mesh,
    out_type=jax.ShapeDtypeStruct((_TOK * _SSTR,), jnp.float32),
    scratch_types=[
        pltpu.VMEM((2, _ICH), jnp.int32),
        pltpu.VMEM((_M + 16,), jnp.int32),
        pltpu.VMEM((_M + 16,), jnp.int32),
        pltpu.VMEM((_M + 16,), jnp.int32),
        pltpu.VMEM((2, _H, _BW), jnp.float32),
        pltpu.VMEM((_H, 64), jnp.float32),
        pltpu.VMEM((_GR, _H), jnp.float32),
        pltpu.SemaphoreType.DMA,
        pltpu.SemaphoreType.DMA,
        pltpu.SemaphoreType.DMA,
    ],
    compiler_params=_params,
)
def _gather_stage(ids_hbm, tab_t_hbm, tail_t_hbm, stage_hbm,
                  idsb_v, ml_v, sup_v, bl_v, blk_v, tail_v, ring_v,
                  sem_ids, sem_blk, sem_out):
    wid = lax.axis_index("s") * _NC + lax.axis_index("c")
    iota = lax.iota(jnp.int32, _LANES)
    cw0 = (wid * _NTC) // _NW
    cw1 = ((wid + 1) * _NTC) // _NW
    cw1p = lax.min(cw1, _NTC - 1)
    ncols = cw1p - cw0
    nblk = (ncols + 3) // 4
    thresh = (_NTC - 1 - cw0) * 128

    def filter_list(src_v, n_src, dst_v, pred):
        def body(q, acc):
            p16 = src_v[pl.ds(q * _LANES, _LANES)]
            valid = (iota + _LANES * q) < n_src
            m = valid & pred(p16 & 0x7FFF)
            pfx = plsc.cumsum(m.astype(jnp.int32))
            iidx = jnp.where(m, acc + pfx - 1, jnp.int32(_M))
            plsc.store_scatter(dst_v, [iidx], p16)
            return acc + plsc.all_reduce_population_count(m)[0]

        return lax.fori_loop(0, (n_src + _LANES - 1) // _LANES, body, 0)

    def extract_scalar(src_v, i):
        chunk = src_v[pl.ds((i // _LANES) * _LANES, _LANES)]
        lane = lax.rem(i, _LANES)
        return chunk[jnp.full((_LANES,), lane, jnp.int32)][0]

    def fire_stage(src_ref, l, n, g):
        r = lax.rem(g, _GR)

        @pl.when(g >= _GR)
        def _():
            pltpu.make_async_copy(
                stage_hbm.at[pl.ds(0, _H)], ring_v.at[0], sem_out
            ).wait()

        lvec = jnp.full((_LANES,), l, jnp.int32)
        for j in range(_H // _LANES):
            ring_v[r, pl.ds(j * _LANES, _LANES)] = plsc.load_gather(
                src_ref, [iota + j * _LANES, lvec]
            )
        pltpu.async_copy(
            ring_v.at[r], stage_hbm.at[pl.ds(n * _SSTR, _H)], sem_out
        )
        return g + 1

    def fire_ids(ci):
        pltpu.async_copy(
            ids_hbm.at[pl.ds(ci * _ICH, _ICH)],
            idsb_v.at[lax.rem(ci, 2)], sem_ids,
        )

    def fire_blk(h):
        off = pl.multiple_of((cw0 + 4 * h) * 128, 128)
        pltpu.async_copy(
            tab_t_hbm.at[:, pl.ds(off, _BW)], blk_v.at[lax.rem(h, 2)],
            sem_blk,
        )

    def do_round(r, g, nch):
        rbase = r * _M

        @pl.when(nch > 0)
        def _():
            fire_ids(0)
            fire_ids(1)

        def chunk_body(ci, cnt):
            pltpu.make_async_copy(
                ids_hbm.at[pl.ds(0, _ICH)], idsb_v.at[0], sem_ids
            ).wait()
            par = lax.rem(ci, 2)

            def scan_body(k, cnt):
                x = idsb_v[par, pl.ds(k * _LANES, _LANES)]
                c = x >> 7
                m = (c >= cw0) & (c < cw1)
                vloc = x - cw0 * 128
                n = (ci * _ICH + k * _LANES) + iota
                p = (n << 15) | vloc
                pfx = plsc.cumsum(m.astype(jnp.int32))
                pos = cnt + pfx - 1 - rbase
                keep = m & (pos >= 0) & (pos < _M)
                iidx = jnp.where(keep, pos, jnp.int32(_M))
                plsc.store_scatter(ml_v, [iidx], p)
                return cnt + plsc.all_reduce_population_count(m)[0]

            cnt = lax.fori_loop(0, _ICH // _LANES, scan_body, cnt)

            @pl.when(ci + 2 < nch)
            def _():
                fire_ids(ci + 2)

            return cnt

        ntot = lax.fori_loop(0, nch, chunk_body, 0)
        n0 = lax.max(lax.min(ntot - rbase, _M), 0)
        nblk_r = jnp.where(n0 > 0, nblk, 0)

        @pl.when(nblk_r > 0)
        def _():
            fire_blk(0)

        for sc in range(_NSUP):
            n_sc = filter_list(ml_v, n0, sup_v,
                               lambda vl, sc=sc: (vl >> 12) == sc)

            def blk_body(hb, g):
                @pl.when(hb + 1 < nblk_r)
                def _():
                    fire_blk(hb + 1)

                pltpu.make_async_copy(
                    tab_t_hbm.at[:, pl.ds(0, _BW)], blk_v.at[0], sem_blk
                ).wait()

                n_b = filter_list(sup_v, n_sc, bl_v,
                                  lambda vl, hb=hb: (vl >> 9) == hb)
                blk = blk_v.at[lax.rem(hb, 2)]

                def match_body(i, g):
                    p = extract_scalar(bl_v, i)
                    return fire_stage(blk, p & 0x1FF, p >> 15, g)

                return lax.fori_loop(0, n_b, match_body, g)

            lo = lax.min(sc * 8, nblk_r)
            hi = lax.min((sc + 1) * 8, nblk_r)
            g = lax.fori_loop(lo, hi, blk_body, g)

        @pl.when((cw1 == _NTC) & (n0 > 0))
        def _():
            pltpu.sync_copy(tail_t_hbm, tail_v)

        n_t = filter_list(ml_v, jnp.where(cw1 == _NTC, n0, 0), bl_v,
                          lambda vl: vl >= thresh)

        def tail_body(i, g):
            p = extract_scalar(bl_v, i)
            return fire_stage(tail_v, (p & 0x7FFF) - thresh, p >> 15, g)

        g = lax.fori_loop(0, n_t, tail_body, g)
        return ntot, g

    ntot, g = do_round(0, 0, jnp.int32(_TOK // _ICH))
    _, g = do_round(
        1, g, jnp.where(ntot > _M, jnp.int32(_TOK // _ICH), jnp.int32(0))
    )

    def drain(i, carry):
        pltpu.make_async_copy(
            stage_hbm.at[pl.ds(0, _H)], ring_v.at[0], sem_out
        ).wait()
        return carry

    lax.fori_loop(0, lax.min(g, _GR), drain, 0)


_C = 512


@functools.partial(
    pl.kernel,
    mesh=_mesh,
    out_type=jax.ShapeDtypeStruct((_B, _H, _S), jnp.float32),
    scratch_types=[
        pltpu.VMEM((_C * _SSTR,), jnp.float32),
        pltpu.VMEM((_H, _C), jnp.float32),
        pltpu.VMEM((_H, _C), jnp.float32),
    ],
    compiler_params=_params,
)
def _pos_add(stage_hbm, pos_t_hbm, out_hbm, stg_v, pos_v, rows_v):
    wid = lax.axis_index("s") * _NC + lax.axis_index("c")
    iota = lax.iota(jnp.int32, _LANES)
    base = wid * _TPW
    b = base // _S
    s0 = base - b * _S
    for ci in range(_TPW // _C):
        coff = ci * _C
        pltpu.sync_copy(
            stage_hbm.at[pl.ds((base + coff) * _SSTR, _C * _SSTR)], stg_v
        )
        pltpu.sync_copy(pos_t_hbm.at[:, pl.ds(s0 + coff, _C)], pos_v)

        def add_body(f, carry):
            for t in range(_C // _LANES):
                sl = pl.ds(t * _LANES, _LANES)
                idx = (iota + t * _LANES) * _SSTR + f
                rows_v[f, sl] = plsc.load_gather(stg_v, [idx]) + pos_v[f, sl]
            return carry

        lax.fori_loop(0, _H, add_body, 0)
        pltpu.sync_copy(rows_v, out_hbm.at[b, :, pl.ds(s0 + coff, _C)])


def kernel(token_ids, embedding_table, pos_table):
    ids = token_ids.reshape(-1)
    tail_t = embedding_table[_VTAIL:].T
    stage = _gather_stage(ids, embedding_table.T, tail_t)
    out3 = _pos_add(stage, pos_table.T)
    return out3.transpose(0, 2, 1)


# speedup vs baseline: 3.1648x; 1.0612x over previous
import functools

import jax
import jax.numpy as jnp
from jax import lax
from jax.experimental import pallas as pl
from jax.experimental.pallas import tpu as pltpu
from jax.experimental.pallas import tpu_sc as plsc

_B = 4
_S = 8192
_H = 64
_V = 1000000
_LANES = 16

_info = plsc.get_sparse_core_info()
_NC = _info.num_cores
_NS = _info.num_subcores
_NW = _NC * _NS

_TOK = _B * _S
_TPW = _TOK // _NW
_NTC = (_V + 127) // 128
_VTAIL = (_NTC - 1) * 128
_NSUP = 8
_GR = 8
_TRASH = _TOK
_SSTR = 72
_M = 16384
_BW = 256
_BD = 4
_ICH = 2048

_mesh = plsc.VectorSubcoreMesh(core_axis_name="c", subcore_axis_name="s")
_params = pltpu.CompilerParams(needs_layout_passes=False)


@functools.partial(
    pl.kernel,
    mesh=_mesh,
    out_type=jax.ShapeDtypeStruct((_TOK * _SSTR,), jnp.float32),
    scratch_types=[
        pltpu.VMEM((2, _ICH), jnp.int32),
        pltpu.VMEM((_M + 16,), jnp.int32),
        pltpu.VMEM((_M + 16,), jnp.int32),
        pltpu.VMEM((_M + 16,), jnp.int32),
        pltpu.VMEM((_BD, _H, _BW), jnp.float32),
        pltpu.VMEM((_H, 64), jnp.float32),
        pltpu.VMEM((_GR, _H), jnp.float32),
        pltpu.SemaphoreType.DMA,
        pltpu.SemaphoreType.DMA,
        pltpu.SemaphoreType.DMA,
    ],
    compiler_params=_params,
)
def _gather_stage(ids_hbm, tab_t_hbm, tail_t_hbm, stage_hbm,
                  idsb_v, ml_v, sup_v, bl_v, blk_v, tail_v, ring_v,
                  sem_ids, sem_blk, sem_out):
    wid = lax.axis_index("s") * _NC + lax.axis_index("c")
    iota = lax.iota(jnp.int32, _LANES)
    cw0 = (wid * _NTC) // _NW
    cw1 = ((wid + 1) * _NTC) // _NW
    cw1p = lax.min(cw1, _NTC - 1)
    ncols = cw1p - cw0
    nblk = (ncols + 1) // 2
    thresh = (_NTC - 1 - cw0) * 128

    def filter_list(src_v, n_src, dst_v, pred):
        def body(q, acc):
            p16 = src_v[pl.ds(q * _LANES, _LANES)]
            valid = (iota + _LANES * q) < n_src
            m = valid & pred(p16 & 0x7FFF)
            pfx = plsc.cumsum(m.astype(jnp.int32))
            iidx = jnp.where(m, acc + pfx - 1, jnp.int32(_M))
            plsc.store_scatter(dst_v, [iidx], p16)
            return acc + plsc.all_reduce_population_count(m)[0]

        return lax.fori_loop(0, (n_src + _LANES - 1) // _LANES, body, 0)

    def extract_scalar(src_v, i):
        chunk = src_v[pl.ds((i // _LANES) * _LANES, _LANES)]
        lane = lax.rem(i, _LANES)
        return chunk[jnp.full((_LANES,), lane, jnp.int32)][0]

    def fire_stage(src_ref, l, n, g):
        r = lax.rem(g, _GR)

        @pl.when(g >= _GR)
        def _():
            pltpu.make_async_copy(
                stage_hbm.at[pl.ds(0, _H)], ring_v.at[0], sem_out
            ).wait()

        lvec = jnp.full((_LANES,), l, jnp.int32)
        for j in range(_H // _LANES):
            ring_v[r, pl.ds(j * _LANES, _LANES)] = plsc.load_gather(
                src_ref, [iota + j * _LANES, lvec]
            )
        pltpu.async_copy(
            ring_v.at[r], stage_hbm.at[pl.ds(n * _SSTR, _H)], sem_out
        )
        return g + 1

    def fire_ids(ci):
        pltpu.async_copy(
            ids_hbm.at[pl.ds(ci * _ICH, _ICH)],
            idsb_v.at[lax.rem(ci, 2)], sem_ids,
        )

    def fire_blk(h):
        off = pl.multiple_of((cw0 + 2 * h) * 128, 128)
        pltpu.async_copy(
            tab_t_hbm.at[:, pl.ds(off, _BW)], blk_v.at[lax.rem(h, _BD)],
            sem_blk,
        )

    def do_round(r, g, nch):
        rbase = r * _M

        @pl.when(nch > 0)
        def _():
            fire_ids(0)
            fire_ids(1)

        def chunk_body(ci, cnt):
            pltpu.make_async_copy(
                ids_hbm.at[pl.ds(0, _ICH)], idsb_v.at[0], sem_ids
            ).wait()
            par = lax.rem(ci, 2)

            def scan_body(k, cnt):
                x = idsb_v[par, pl.ds(k * _LANES, _LANES)]
                c = x >> 7
                m = (c >= cw0) & (c < cw1)
                vloc = x - cw0 * 128
                n = (ci * _ICH + k * _LANES) + iota
                p = (n << 15) | vloc
                pfx = plsc.cumsum(m.astype(jnp.int32))
                pos = cnt + pfx - 1 - rbase
                keep = m & (pos >= 0) & (pos < _M)
                iidx = jnp.where(keep, pos, jnp.int32(_M))
                plsc.store_scatter(ml_v, [iidx], p)
                return cnt + plsc.all_reduce_population_count(m)[0]

            cnt = lax.fori_loop(0, _ICH // _LANES, scan_body, cnt)

            @pl.when(ci + 2 < nch)
            def _():
                fire_ids(ci + 2)

            return cnt

        ntot = lax.fori_loop(0, nch, chunk_body, 0)
        n0 = lax.max(lax.min(ntot - rbase, _M), 0)
        nblk_r = jnp.where(n0 > 0, nblk, 0)

        for pre in range(_BD - 1):
            @pl.when(nblk_r > pre)
            def _(pre=pre):
                fire_blk(pre)

        for sc in range(_NSUP):
            n_sc = filter_list(ml_v, n0, sup_v,
                               lambda vl, sc=sc: (vl >> 12) == sc)

            def blk_body(hb, g):
                @pl.when(hb + _BD - 1 < nblk_r)
                def _():
                    fire_blk(hb + _BD - 1)

                n_b = filter_list(sup_v, n_sc, bl_v,
                                  lambda vl, hb=hb: (vl >> 8) == hb)
                pltpu.make_async_copy(
                    tab_t_hbm.at[:, pl.ds(0, _BW)], blk_v.at[0], sem_blk
                ).wait()
                blk = blk_v.at[lax.rem(hb, _BD)]

                def match_body(i, g):
                    p = extract_scalar(bl_v, i)
                    return fire_stage(blk, p & 0xFF, p >> 15, g)

                return lax.fori_loop(0, n_b, match_body, g)

            lo = lax.min(sc * 16, nblk_r)
            hi = lax.min((sc + 1) * 16, nblk_r)
            g = lax.fori_loop(lo, hi, blk_body, g)

        @pl.when((cw1 == _NTC) & (n0 > 0))
        def _():
            pltpu.sync_copy(tail_t_hbm, tail_v)

        n_t = filter_list(ml_v, jnp.where(cw1 == _NTC, n0, 0), bl_v,
                          lambda vl: vl >= thresh)

        def tail_body(i, g):
            p = extract_scalar(bl_v, i)
            return fire_stage(tail_v, (p & 0x7FFF) - thresh, p >> 15, g)

        g = lax.fori_loop(0, n_t, tail_body, g)
        return ntot, g

    ntot, g = do_round(0, 0, jnp.int32(_TOK // _ICH))
    _, g = do_round(
        1, g, jnp.where(ntot > _M, jnp.int32(_TOK // _ICH), jnp.int32(0))
    )

    def drain(i, carry):
        pltpu.make_async_copy(
            stage_hbm.at[pl.ds(0, _H)], ring_v.at[0], sem_out
        ).wait()
        return carry

    lax.fori_loop(0, lax.min(g, _GR), drain, 0)


_C = 512


@functools.partial(
    pl.kernel,
    mesh=_mesh,
    out_type=jax.ShapeDtypeStruct((_B, _H, _S), jnp.float32),
    scratch_types=[
        pltpu.VMEM((_C * _SSTR,), jnp.float32),
        pltpu.VMEM((_H, _C), jnp.float32),
        pltpu.VMEM((_H, _C), jnp.float32),
    ],
    compiler_params=_params,
)
def _pos_add(stage_hbm, pos_t_hbm, out_hbm, stg_v, pos_v, rows_v):
    wid = lax.axis_index("s") * _NC + lax.axis_index("c")
    iota = lax.iota(jnp.int32, _LANES)
    base = wid * _TPW
    b = base // _S
    s0 = base - b * _S
    for ci in range(_TPW // _C):
        coff = ci * _C
        pltpu.sync_copy(
            stage_hbm.at[pl.ds((base + coff) * _SSTR, _C * _SSTR)], stg_v
        )
        pltpu.sync_copy(pos_t_hbm.at[:, pl.ds(s0 + coff, _C)], pos_v)

        def add_body(f, carry):
            for t in range(_C // _LANES):
                sl = pl.ds(t * _LANES, _LANES)
                idx = (iota + t * _LANES) * _SSTR + f
                rows_v[f, sl] = plsc.load_gather(stg_v, [idx]) + pos_v[f, sl]
            return carry

        lax.fori_loop(0, _H, add_body, 0)
        pltpu.sync_copy(rows_v, out_hbm.at[b, :, pl.ds(s0 + coff, _C)])


def kernel(token_ids, embedding_table, pos_table):
    ids = token_ids.reshape(-1)
    tail_t = embedding_table[_VTAIL:].T
    stage = _gather_stage(ids, embedding_table.T, tail_t)
    out3 = _pos_add(stage, pos_table.T)
    return out3.transpose(0, 2, 1)


# speedup vs baseline: 3.2387x; 1.0233x over previous
import functools

import jax
import jax.numpy as jnp
from jax import lax
from jax.experimental import pallas as pl
from jax.experimental.pallas import tpu as pltpu
from jax.experimental.pallas import tpu_sc as plsc

_B = 4
_S = 8192
_H = 64
_V = 1000000
_LANES = 16

_info = plsc.get_sparse_core_info()
_NC = _info.num_cores
_NS = _info.num_subcores
_NW = _NC * _NS

_TOK = _B * _S
_TPW = _TOK // _NW
_NTC = (_V + 127) // 128
_VTAIL = (_NTC - 1) * 128
_NSUP = 8
_GR = 8
_TRASH = _TOK
_SSTR = 72
_M = 16384
_BW = 256
_BD = 4
_ICH = 2048

_mesh = plsc.VectorSubcoreMesh(core_axis_name="c", subcore_axis_name="s")
_params = pltpu.CompilerParams(needs_layout_passes=False)


@functools.partial(
    pl.kernel,
    mesh=_mesh,
    out_type=jax.ShapeDtypeStruct((_TOK * _SSTR,), jnp.float32),
    scratch_types=[
        pltpu.VMEM((2, _ICH), jnp.int32),
        pltpu.VMEM((_M + 16,), jnp.int32),
        pltpu.VMEM((_M + 16,), jnp.int32),
        pltpu.VMEM((_M + 16,), jnp.int32),
        pltpu.VMEM((_BD, _H, _BW), jnp.float32),
        pltpu.VMEM((_H, 64), jnp.float32),
        pltpu.VMEM((_GR, _H), jnp.float32),
        pltpu.SemaphoreType.DMA,
        pltpu.SemaphoreType.DMA,
        pltpu.SemaphoreType.DMA,
    ],
    compiler_params=_params,
)
def _gather_stage(ids_hbm, tab_t_hbm, tail_t_hbm, stage_hbm,
                  idsb_v, ml_v, sup_v, bl_v, blk_v, tail_v, ring_v,
                  sem_ids, sem_blk, sem_out):
    wid = lax.axis_index("s") * _NC + lax.axis_index("c")
    iota = lax.iota(jnp.int32, _LANES)
    cw0 = (wid * _NTC) // _NW
    cw1 = ((wid + 1) * _NTC) // _NW
    cw1p = lax.min(cw1, _NTC - 1)
    ncols = cw1p - cw0
    nblk = (ncols + 1) // 2
    thresh = (_NTC - 1 - cw0) * 128

    def filter_list(src_v, n_src, dst_v, pred):
        def body(q, acc):
            p16 = src_v[pl.ds(q * _LANES, _LANES)]
            valid = (iota + _LANES * q) < n_src
            m = valid & pred(p16 & 0x7FFF)
            pfx = plsc.cumsum(m.astype(jnp.int32))
            iidx = jnp.where(m, acc + pfx - 1, jnp.int32(_M))
            plsc.store_scatter(dst_v, [iidx], p16)
            return acc + plsc.all_reduce_population_count(m)[0]

        return lax.fori_loop(0, (n_src + _LANES - 1) // _LANES, body, 0)

    def extract_scalar(src_v, i):
        chunk = src_v[pl.ds((i // _LANES) * _LANES, _LANES)]
        lane = lax.rem(i, _LANES)
        return chunk[jnp.full((_LANES,), lane, jnp.int32)][0]

    def fire_stage(src_ref, l, n, g):
        r = lax.rem(g, _GR)

        @pl.when(g >= _GR)
        def _():
            pltpu.make_async_copy(
                stage_hbm.at[pl.ds(0, _H)], ring_v.at[0], sem_out
            ).wait()

        lvec = jnp.full((_LANES,), l, jnp.int32)
        for j in range(_H // _LANES):
            ring_v[r, pl.ds(j * _LANES, _LANES)] = plsc.load_gather(
                src_ref, [iota + j * _LANES, lvec]
            )
        pltpu.async_copy(
            ring_v.at[r], stage_hbm.at[pl.ds(n * _SSTR, _H)], sem_out
        )
        return g + 1

    def fire_ids(ci):
        pltpu.async_copy(
            ids_hbm.at[pl.ds(ci * _ICH, _ICH)],
            idsb_v.at[lax.rem(ci, 2)], sem_ids,
        )

    def fire_blk(h):
        off = pl.multiple_of((cw0 + 2 * h) * 128, 128)
        pltpu.async_copy(
            tab_t_hbm.at[:, pl.ds(off, _BW)], blk_v.at[lax.rem(h, _BD)],
            sem_blk,
        )

    def do_round(r, g, nch):
        rbase = r * _M

        @pl.when(nch > 0)
        def _():
            fire_ids(0)
            fire_ids(1)

        def chunk_body(ci, cnt):
            pltpu.make_async_copy(
                ids_hbm.at[pl.ds(0, _ICH)], idsb_v.at[0], sem_ids
            ).wait()
            par = lax.rem(ci, 2)

            def scan_body(k, cnt):
                x = idsb_v[par, pl.ds(k * _LANES, _LANES)]
                c = x >> 7
                m = (c >= cw0) & (c < cw1)
                vloc = x - cw0 * 128
                n = (ci * _ICH + k * _LANES) + iota
                p = (n << 15) | vloc
                pfx = plsc.cumsum(m.astype(jnp.int32))
                pos = cnt + pfx - 1 - rbase
                keep = m & (pos >= 0) & (pos < _M)
                iidx = jnp.where(keep, pos, jnp.int32(_M))
                plsc.store_scatter(ml_v, [iidx], p)
                return cnt + plsc.all_reduce_population_count(m)[0]

            cnt = lax.fori_loop(0, _ICH // _LANES, scan_body, cnt)

            @pl.when(ci + 2 < nch)
            def _():
                fire_ids(ci + 2)

            return cnt

        ntot = lax.fori_loop(0, nch, chunk_body, 0)
        n0 = lax.max(lax.min(ntot - rbase, _M), 0)
        nblk_r = jnp.where(n0 > 0, nblk, 0)

        for pre in range(_BD - 1):
            @pl.when(nblk_r > pre)
            def _(pre=pre):
                fire_blk(pre)

        for sc in range(_NSUP):
            n_sc = filter_list(ml_v, n0, sup_v,
                               lambda vl, sc=sc: (vl >> 12) == sc)

            def blk_body(hb, g):
                @pl.when(hb + _BD - 1 < nblk_r)
                def _():
                    fire_blk(hb + _BD - 1)

                n_b = filter_list(sup_v, n_sc, bl_v,
                                  lambda vl, hb=hb: (vl >> 8) == hb)
                pltpu.make_async_copy(
                    tab_t_hbm.at[:, pl.ds(0, _BW)], blk_v.at[0], sem_blk
                ).wait()
                blk = blk_v.at[lax.rem(hb, _BD)]

                def match_body(i, g):
                    p = extract_scalar(bl_v, i)
                    return fire_stage(blk, p & 0xFF, p >> 15, g)

                return lax.fori_loop(0, n_b, match_body, g)

            lo = lax.min(sc * 16, nblk_r)
            hi = lax.min((sc + 1) * 16, nblk_r)
            g = lax.fori_loop(lo, hi, blk_body, g)

        @pl.when((cw1 == _NTC) & (n0 > 0))
        def _():
            pltpu.sync_copy(tail_t_hbm, tail_v)

        n_t = filter_list(ml_v, jnp.where(cw1 == _NTC, n0, 0), bl_v,
                          lambda vl: vl >= thresh)

        def tail_body(i, g):
            p = extract_scalar(bl_v, i)
            return fire_stage(tail_v, (p & 0x7FFF) - thresh, p >> 15, g)

        g = lax.fori_loop(0, n_t, tail_body, g)
        return ntot, g

    ntot, g = do_round(0, 0, jnp.int32(_TOK // _ICH))
    _, g = do_round(
        1, g, jnp.where(ntot > _M, jnp.int32(_TOK // _ICH), jnp.int32(0))
    )

    def drain(i, carry):
        pltpu.make_async_copy(
            stage_hbm.at[pl.ds(0, _H)], ring_v.at[0], sem_out
        ).wait()
        return carry

    lax.fori_loop(0, lax.min(g, _GR), drain, 0)


_C = 256


@functools.partial(
    pl.kernel,
    mesh=_mesh,
    out_type=jax.ShapeDtypeStruct((_B, _H, _S), jnp.float32),
    scratch_types=[
        pltpu.VMEM((2, _C * _SSTR), jnp.float32),
        pltpu.VMEM((2, _H, _C), jnp.float32),
        pltpu.VMEM((2, _H, _C), jnp.float32),
        pltpu.SemaphoreType.DMA,
        pltpu.SemaphoreType.DMA,
    ],
    compiler_params=_params,
)
def _pos_add(stage_hbm, pos_t_hbm, out_hbm, stg_v, pos_v, rows_v,
             sem_in, sem_out):
    wid = lax.axis_index("s") * _NC + lax.axis_index("c")
    iota = lax.iota(jnp.int32, _LANES)
    base = wid * _TPW
    b = base // _S
    s0 = base - b * _S
    nch = _TPW // _C

    def fire_in(ci):
        par = ci % 2
        pltpu.async_copy(
            stage_hbm.at[pl.ds((base + ci * _C) * _SSTR, _C * _SSTR)],
            stg_v.at[par], sem_in,
        )
        pltpu.async_copy(
            pos_t_hbm.at[:, pl.ds(s0 + ci * _C, _C)], pos_v.at[par], sem_in
        )

    fire_in(0)
    for ci in range(nch):
        par = ci % 2
        if ci + 1 < nch:
            fire_in(ci + 1)
        pltpu.make_async_copy(
            stage_hbm.at[pl.ds(0, _C * _SSTR)], stg_v.at[0], sem_in
        ).wait()
        pltpu.make_async_copy(
            pos_t_hbm.at[:, pl.ds(0, _C)], pos_v.at[0], sem_in
        ).wait()
        if ci >= 2:
            pltpu.make_async_copy(
                rows_v.at[0], out_hbm.at[b, :, pl.ds(0, _C)], sem_out
            ).wait()

        def add_body(f, carry):
            for t in range(_C // _LANES):
                sl = pl.ds(t * _LANES, _LANES)
                idx = (iota + t * _LANES) * _SSTR + f
                pvec = jnp.full((_LANES,), par, jnp.int32)
                rows_v[par, f, sl] = (
                    plsc.load_gather(stg_v, [pvec, idx]) + pos_v[par, f, sl]
                )
            return carry

        lax.fori_loop(0, _H, add_body, 0)
        pltpu.async_copy(
            rows_v.at[par], out_hbm.at[b, :, pl.ds(s0 + ci * _C, _C)],
            sem_out,
        )

    for _ in range(min(nch, 2)):
        pltpu.make_async_copy(
            rows_v.at[0], out_hbm.at[b, :, pl.ds(0, _C)], sem_out
        ).wait()


def kernel(token_ids, embedding_table, pos_table):
    ids = token_ids.reshape(-1)
    tail_t = embedding_table[_VTAIL:].T
    stage = _gather_stage(ids, embedding_table.T, tail_t)
    out3 = _pos_add(stage, pos_table.T)
    return out3.transpose(0, 2, 1)
